# baseline (speedup 1.0000x reference)
import functools

import jax
import jax.numpy as jnp
from jax import lax
from jax.experimental import pallas as pl
from jax.experimental.pallas import tpu as pltpu
from jax.experimental.pallas import tpu_sc as plsc

N_NODES = 10000
N_EDGES = 320000
D_FEAT = 128
HIDDEN = 8
OUT2 = 2
N_CLASSES = 4

NC = 2
NS = 16
NW = NC * NS
BLK = 128
NBW = N_EDGES // BLK
NB0 = NBW // NW
XTRA = NBW - NB0 * NW
P = 10240
RPT = P // NS
RPN = N_NODES // NS

_mesh = plsc.VectorSubcoreMesh(
    core_axis_name="c", subcore_axis_name="s", num_cores=NC, num_subcores=NS
)
_cp = pltpu.CompilerParams(use_tc_tiling_on_sc=False)


@functools.partial(
    pl.kernel,
    out_type=jax.ShapeDtypeStruct((NC * P,), jnp.float32),
    mesh=_mesh,
    compiler_params=_cp,
    scratch_types=[
        pltpu.VMEM((NB0 + 1, BLK), jnp.int32),
        pltpu.VMEM((BLK,), jnp.float32),
        pltpu.VMEM_SHARED((P,), jnp.float32),
        pltpu.SemaphoreType.DMA,
        pltpu.SemaphoreType.DMA,
    ],
)
def _deg_kernel(e_hbm, z_hbm, out_hbm, di, ones, hist, hs0, hs1):
    c = lax.axis_index("c")
    s = lax.axis_index("s")
    w = c * NS + s
    pltpu.sync_copy(z_hbm.at[pl.ds(s * RPT, RPT)], hist.at[pl.ds(s * RPT, RPT)])
    for i in range(BLK // 16):
        ones[pl.ds(i * 16, 16)] = jnp.ones((16,), jnp.float32)
    pltpu.sync_copy(e_hbm.at[1, pl.ds(w * NB0, NB0)], di.at[pl.ds(0, NB0)])

    @pl.when(w < XTRA)
    def _():
        pltpu.sync_copy(e_hbm.at[1, pl.ds(NW * NB0 + w, 1)], di.at[pl.ds(NB0, 1)])

    plsc.subcore_barrier()

    def body(i, carry):
        d0 = pltpu.async_copy(ones, hist.at[di.at[2 * i]], hs0, add=True)
        d1 = pltpu.async_copy(ones, hist.at[di.at[2 * i + 1]], hs1, add=True)
        d0.wait()
        d1.wait()
        return carry

    lax.fori_loop(0, NB0 // 2, body, 0)

    @pl.when(w < XTRA)
    def _():
        pltpu.sync_copy(ones, hist.at[di.at[NB0]], add=True)

    plsc.subcore_barrier()
    pltpu.sync_copy(
        hist.at[pl.ds(s * RPT, RPT)], out_hbm.at[pl.ds(c * P + s * RPT, RPT)]
    )


def _make_scatter(D):
    @functools.partial(
        pl.kernel,
        out_type=jax.ShapeDtypeStruct((NC, N_NODES, D), jnp.float32),
        mesh=_mesh,
        compiler_params=_cp,
        scratch_types=[
            pltpu.VMEM((NB0 + 1, BLK), jnp.int32),
            pltpu.VMEM((NB0 + 1, BLK), jnp.int32),
            pltpu.VMEM((4, BLK, D), jnp.float32),
            pltpu.VMEM_SHARED((N_NODES, D), jnp.float32),
            pltpu.VMEM_SHARED((N_NODES, D), jnp.float32),
            pltpu.SemaphoreType.DMA,
            pltpu.SemaphoreType.DMA,
            pltpu.SemaphoreType.DMA,
            pltpu.SemaphoreType.DMA,
            pltpu.SemaphoreType.DMA,
            pltpu.SemaphoreType.DMA,
            pltpu.SemaphoreType.DMA,
            pltpu.SemaphoreType.DMA,
        ],
    )
    def scat(e_hbm, y_hbm, z_hbm, out_hbm, si, di, rows, ytab, agg,
             gs0, gs1, gs2, gs3, ss0, ss1, ss2, ss3):
        c = lax.axis_index("c")
        s = lax.axis_index("s")
        w = c * NS + s
        pltpu.sync_copy(z_hbm.at[pl.ds(s * RPN, RPN)], agg.at[pl.ds(s * RPN, RPN)])
        pltpu.sync_copy(y_hbm.at[pl.ds(s * RPN, RPN)], ytab.at[pl.ds(s * RPN, RPN)])
        pltpu.sync_copy(e_hbm.at[0, pl.ds(w * NB0, NB0)], si.at[pl.ds(0, NB0)])
        pltpu.sync_copy(e_hbm.at[1, pl.ds(w * NB0, NB0)], di.at[pl.ds(0, NB0)])

        @pl.when(w < XTRA)
        def _():
            pltpu.sync_copy(e_hbm.at[0, pl.ds(NW * NB0 + w, 1)], si.at[pl.ds(NB0, 1)])
            pltpu.sync_copy(e_hbm.at[1, pl.ds(NW * NB0 + w, 1)], di.at[pl.ds(NB0, 1)])

        plsc.subcore_barrier()
        gsems = (gs0, gs1, gs2, gs3)
        ssems = (ss0, ss1, ss2, ss3)

        def body(i, carry):
            j = 4 * i
            gd = [
                pltpu.async_copy(ytab.at[si.at[j + k]], rows.at[k], gsems[k])
                for k in range(4)
            ]
            sd = []
            for k in range(4):
                gd[k].wait()
                sd.append(
                    pltpu.async_copy(rows.at[k], agg.at[di.at[j + k]], ssems[k],
                                     add=True)
                )
            for k in range(4):
                sd[k].wait()
            return carry

        lax.fori_loop(0, NB0 // 4, body, 0)

        t0 = NB0 - 2
        d0 = pltpu.async_copy(ytab.at[si.at[t0]], rows.at[0], gs0)
        d1 = pltpu.async_copy(ytab.at[si.at[t0 + 1]], rows.at[1], gs1)
        d0.wait()
        s0 = pltpu.async_copy(rows.at[0], agg.at[di.at[t0]], ss0, add=True)
        d1.wait()
        s1 = pltpu.async_copy(rows.at[1], agg.at[di.at[t0 + 1]], ss1, add=True)
        s0.wait()
        s1.wait()

        @pl.when(w < XTRA)
        def _():
            pltpu.async_copy(ytab.at[si.at[NB0]], rows.at[2], gs2).wait()
            pltpu.sync_copy(rows.at[2], agg.at[di.at[NB0]], add=True)

        plsc.subcore_barrier()
        pltpu.sync_copy(agg.at[pl.ds(s * RPN, RPN)], out_hbm.at[c, pl.ds(s * RPN, RPN)])

    return scat


_scatter8 = _make_scatter(HIDDEN)


def _tcxw_body(x_ref, w_ref, xw_ref):
    xw_ref[...] = jnp.dot(
        x_ref[...], w_ref[...], preferred_element_type=jnp.float32
    )


_tcxw = pl.pallas_call(
    _tcxw_body,
    out_shape=jax.ShapeDtypeStruct((N_NODES, HIDDEN), jnp.float32),
)


def _tc1_body(xw_ref, hist_ref, y_ref):
    h = hist_ref[...]
    dis = lax.rsqrt(h[0:P] + h[P:2 * P] + 1.0)[:N_NODES, None]
    y_ref[...] = xw_ref[...] * dis


_tc1 = pl.pallas_call(
    _tc1_body,
    out_shape=jax.ShapeDtypeStruct((N_NODES, HIDDEN), jnp.float32),
)


def _tc2_body(hist_ref, agg_ref, y1_ref, b1_ref, w2_ref, y2_ref):
    h = hist_ref[...]
    dis = lax.rsqrt(h[0:P] + h[P:2 * P] + 1.0)[:N_NODES, None]
    a = agg_ref[...]
    s = a[0] + a[1] + y1_ref[...]
    hh = jnp.tanh(dis * s + b1_ref[...])
    y2_ref[...] = jnp.dot(hh, w2_ref[...], preferred_element_type=jnp.float32) * dis


_tc2 = pl.pallas_call(
    _tc2_body,
    out_shape=jax.ShapeDtypeStruct((N_NODES, HIDDEN), jnp.float32),
)


def _tc3_body(hist_ref, agg_ref, y2_ref, b2_ref, wc_ref, bc_ref, out_ref, h2_ref):
    h = hist_ref[...]
    dis = lax.rsqrt(h[0:P] + h[P:2 * P] + 1.0)[:N_NODES, None]
    a = agg_ref[...]
    s = a[0] + a[1] + y2_ref[...]
    hh = jnp.tanh(dis * s + b2_ref[...])
    h2 = hh[:, :OUT2]
    h2_ref[...] = h2
    out_ref[...] = (
        jnp.dot(h2, wc_ref[...], preferred_element_type=jnp.float32) + bc_ref[...]
    )


_tc3 = pl.pallas_call(
    _tc3_body,
    out_shape=[
        jax.ShapeDtypeStruct((N_NODES, N_CLASSES), jnp.float32),
        jax.ShapeDtypeStruct((N_NODES, OUT2), jnp.float32),
    ],
)


def kernel(x, edge_index, W1, b1, W2, b2, Wc, bc):
    f32 = jnp.float32
    e3 = edge_index.reshape(2, NBW, BLK)
    z1 = jnp.zeros((P,), f32)
    z8 = jnp.zeros((N_NODES, HIDDEN), f32)
    w2p = jnp.pad(W2, ((0, 0), (0, HIDDEN - OUT2)))
    b2p = jnp.pad(b2, (0, HIDDEN - OUT2)).reshape(1, HIDDEN)

    xw = _tcxw(x, W1)
    hist = _deg_kernel(e3, z1)
    y1 = _tc1(xw, hist)
    agg1 = _scatter8(e3, y1, z8)
    y2 = _tc2(hist, agg1, y1, b1.reshape(1, HIDDEN), w2p)
    agg2 = _scatter8(e3, y2, z8)
    out, h2 = _tc3(hist, agg2, y2, b2p, Wc, bc.reshape(1, N_CLASSES))
    return (out, h2)


# speedup vs baseline: 1.0411x; 1.0411x over previous
import functools

import jax
import jax.numpy as jnp
from jax import lax
from jax.experimental import pallas as pl
from jax.experimental.pallas import tpu as pltpu
from jax.experimental.pallas import tpu_sc as plsc

N_NODES = 10000
N_EDGES = 320000
D_FEAT = 128
HIDDEN = 8
OUT2 = 2
N_CLASSES = 4

NC = 2
NS = 16
NW = NC * NS
BLK = 128
NBW = N_EDGES // BLK
NB0 = NBW // NW
XTRA = NBW - NB0 * NW
P = 10240
RPT = P // NS
RPN = N_NODES // NS

_mesh = plsc.VectorSubcoreMesh(
    core_axis_name="c", subcore_axis_name="s", num_cores=NC, num_subcores=NS
)
_cp = pltpu.CompilerParams(use_tc_tiling_on_sc=False)


@functools.partial(
    pl.kernel,
    out_type=jax.ShapeDtypeStruct((NC * P,), jnp.float32),
    mesh=_mesh,
    compiler_params=_cp,
    scratch_types=[
        pltpu.VMEM((NB0 + 1, BLK), jnp.int32),
        pltpu.VMEM((BLK,), jnp.float32),
        pltpu.VMEM_SHARED((P,), jnp.float32),
        pltpu.SemaphoreType.DMA,
        pltpu.SemaphoreType.DMA,
    ],
)
def _deg_kernel(e_hbm, z_hbm, out_hbm, di, ones, hist, hs0, hs1):
    c = lax.axis_index("c")
    s = lax.axis_index("s")
    w = c * NS + s
    pltpu.sync_copy(z_hbm.at[pl.ds(s * RPT, RPT)], hist.at[pl.ds(s * RPT, RPT)])
    for i in range(BLK // 16):
        ones[pl.ds(i * 16, 16)] = jnp.ones((16,), jnp.float32)
    pltpu.sync_copy(e_hbm.at[1, pl.ds(w * NB0, NB0)], di.at[pl.ds(0, NB0)])

    @pl.when(w < XTRA)
    def _():
        pltpu.sync_copy(e_hbm.at[1, pl.ds(NW * NB0 + w, 1)], di.at[pl.ds(NB0, 1)])

    plsc.subcore_barrier()

    def body(i, carry):
        d0 = pltpu.async_copy(ones, hist.at[di.at[2 * i]], hs0, add=True)
        d1 = pltpu.async_copy(ones, hist.at[di.at[2 * i + 1]], hs1, add=True)
        d0.wait()
        d1.wait()
        return carry

    lax.fori_loop(0, NB0 // 2, body, 0)

    @pl.when(w < XTRA)
    def _():
        pltpu.sync_copy(ones, hist.at[di.at[NB0]], add=True)

    plsc.subcore_barrier()
    pltpu.sync_copy(
        hist.at[pl.ds(s * RPT, RPT)], out_hbm.at[pl.ds(c * P + s * RPT, RPT)]
    )


def _make_scatter(D):
    @functools.partial(
        pl.kernel,
        out_type=jax.ShapeDtypeStruct((NC, N_NODES, D), jnp.float32),
        mesh=_mesh,
        compiler_params=_cp,
        scratch_types=[
            pltpu.VMEM((NB0 + 1, BLK), jnp.int32),
            pltpu.VMEM((NB0 + 1, BLK), jnp.int32),
            pltpu.VMEM((4, BLK, D), jnp.float32),
            pltpu.VMEM_SHARED((N_NODES, D), jnp.float32),
            pltpu.VMEM_SHARED((N_NODES, D), jnp.float32),
            pltpu.SemaphoreType.DMA,
            pltpu.SemaphoreType.DMA,
            pltpu.SemaphoreType.DMA,
            pltpu.SemaphoreType.DMA,
            pltpu.SemaphoreType.DMA,
            pltpu.SemaphoreType.DMA,
            pltpu.SemaphoreType.DMA,
            pltpu.SemaphoreType.DMA,
        ],
    )
    def scat(e_hbm, y_hbm, z_hbm, out_hbm, si, di, rows, ytab, agg,
             gs0, gs1, gs2, gs3, ss0, ss1, ss2, ss3):
        c = lax.axis_index("c")
        s = lax.axis_index("s")
        w = c * NS + s
        p0 = pltpu.async_copy(
            z_hbm.at[pl.ds(s * RPN, RPN)], agg.at[pl.ds(s * RPN, RPN)], gs0)
        p1 = pltpu.async_copy(
            y_hbm.at[pl.ds(s * RPN, RPN)], ytab.at[pl.ds(s * RPN, RPN)], gs1)
        p2 = pltpu.async_copy(
            e_hbm.at[0, pl.ds(w * NB0, NB0)], si.at[pl.ds(0, NB0)], gs2)
        p3 = pltpu.async_copy(
            e_hbm.at[1, pl.ds(w * NB0, NB0)], di.at[pl.ds(0, NB0)], gs3)
        p0.wait()
        p1.wait()
        p2.wait()
        p3.wait()

        @pl.when(w < XTRA)
        def _():
            pltpu.sync_copy(e_hbm.at[0, pl.ds(NW * NB0 + w, 1)], si.at[pl.ds(NB0, 1)])
            pltpu.sync_copy(e_hbm.at[1, pl.ds(NW * NB0 + w, 1)], di.at[pl.ds(NB0, 1)])

        plsc.subcore_barrier()
        gsems = (gs0, gs1, gs2, gs3)
        ssems = (ss0, ss1, ss2, ss3)

        def body(i, carry):
            j = 4 * i
            gd = [
                pltpu.async_copy(ytab.at[si.at[j + k]], rows.at[k], gsems[k])
                for k in range(4)
            ]
            sd = []
            for k in range(4):
                gd[k].wait()
                sd.append(
                    pltpu.async_copy(rows.at[k], agg.at[di.at[j + k]], ssems[k],
                                     add=True)
                )
            for k in range(4):
                sd[k].wait()
            return carry

        lax.fori_loop(0, NB0 // 4, body, 0)

        t0 = NB0 - 2
        d0 = pltpu.async_copy(ytab.at[si.at[t0]], rows.at[0], gs0)
        d1 = pltpu.async_copy(ytab.at[si.at[t0 + 1]], rows.at[1], gs1)
        d0.wait()
        s0 = pltpu.async_copy(rows.at[0], agg.at[di.at[t0]], ss0, add=True)
        d1.wait()
        s1 = pltpu.async_copy(rows.at[1], agg.at[di.at[t0 + 1]], ss1, add=True)
        s0.wait()
        s1.wait()

        @pl.when(w < XTRA)
        def _():
            pltpu.async_copy(ytab.at[si.at[NB0]], rows.at[2], gs2).wait()
            pltpu.sync_copy(rows.at[2], agg.at[di.at[NB0]], add=True)

        plsc.subcore_barrier()
        pltpu.sync_copy(agg.at[pl.ds(s * RPN, RPN)], out_hbm.at[c, pl.ds(s * RPN, RPN)])

    return scat


_scatter8 = _make_scatter(HIDDEN)


def _tc1_body(x_ref, w_ref, hist_ref, y_ref):
    h = hist_ref[...]
    dis = lax.rsqrt(h[0:P] + h[P:2 * P] + 1.0)[:N_NODES, None]
    xw = jnp.dot(x_ref[...], w_ref[...], preferred_element_type=jnp.float32)
    y_ref[...] = xw * dis


_tc1 = pl.pallas_call(
    _tc1_body,
    out_shape=jax.ShapeDtypeStruct((N_NODES, HIDDEN), jnp.float32),
)


def _tc2_body(hist_ref, agg_ref, y1_ref, b1_ref, w2_ref, y2_ref):
    h = hist_ref[...]
    dis = lax.rsqrt(h[0:P] + h[P:2 * P] + 1.0)[:N_NODES, None]
    a = agg_ref[...]
    s = a[0] + a[1] + y1_ref[...]
    hh = jnp.tanh(dis * s + b1_ref[...])
    y2_ref[...] = jnp.dot(hh, w2_ref[...], preferred_element_type=jnp.float32) * dis


_tc2 = pl.pallas_call(
    _tc2_body,
    out_shape=jax.ShapeDtypeStruct((N_NODES, HIDDEN), jnp.float32),
)


def _tc3_body(hist_ref, agg_ref, y2_ref, b2_ref, wc_ref, bc_ref, out_ref, h2_ref):
    h = hist_ref[...]
    dis = lax.rsqrt(h[0:P] + h[P:2 * P] + 1.0)[:N_NODES, None]
    a = agg_ref[...]
    s = a[0] + a[1] + y2_ref[...]
    hh = jnp.tanh(dis * s + b2_ref[...])
    h2 = hh[:, :OUT2]
    h2_ref[...] = h2
    out_ref[...] = (
        jnp.dot(h2, wc_ref[...], preferred_element_type=jnp.float32) + bc_ref[...]
    )


_tc3 = pl.pallas_call(
    _tc3_body,
    out_shape=[
        jax.ShapeDtypeStruct((N_NODES, N_CLASSES), jnp.float32),
        jax.ShapeDtypeStruct((N_NODES, OUT2), jnp.float32),
    ],
)


def kernel(x, edge_index, W1, b1, W2, b2, Wc, bc):
    f32 = jnp.float32
    e3 = edge_index.reshape(2, NBW, BLK)
    z1 = jnp.zeros((P,), f32)
    z8 = jnp.zeros((N_NODES, HIDDEN), f32)
    w2p = jnp.pad(W2, ((0, 0), (0, HIDDEN - OUT2)))
    b2p = jnp.pad(b2, (0, HIDDEN - OUT2)).reshape(1, HIDDEN)

    hist = _deg_kernel(e3, z1)
    y1 = _tc1(x, W1, hist)
    agg1 = _scatter8(e3, y1, z8)
    y2 = _tc2(hist, agg1, y1, b1.reshape(1, HIDDEN), w2p)
    agg2 = _scatter8(e3, y2, z8)
    out, h2 = _tc3(hist, agg2, y2, b2p, Wc, bc.reshape(1, N_CLASSES))
    return (out, h2)
